# baseline (speedup 1.0000x reference)
import functools

import jax
import jax.numpy as jnp
from jax import lax
from jax.experimental import pallas as pl
from jax.experimental.pallas import tpu as pltpu, tpu_sc as plsc

_L = 16


def _sc_encode(boxes_t, gtb_t, gtf_t, gid2, flag2, B, N, G):
    info = plsc.get_sparse_core_info()
    NC, NS = info.num_cores, info.num_subcores
    assert NC * NS == B, (NC, NS, B)
    n_full = N // _L
    has_tail = (N % _L) != 0
    mesh = plsc.VectorSubcoreMesh(core_axis_name="c", subcore_axis_name="s")

    @functools.partial(
        pl.kernel,
        out_type=tuple(jax.ShapeDtypeStruct((B, N), jnp.float32)
                       for _ in range(5)),
        mesh=mesh,
        compiler_params=pltpu.CompilerParams(
            needs_layout_passes=False,
            skip_device_barrier=True,
            disable_bounds_checks=True,
            disable_semaphore_checks=True,
        ),
        scratch_types=[
            pltpu.VMEM((N,), jnp.float32),
            pltpu.VMEM((N,), jnp.float32),
            pltpu.VMEM((N,), jnp.float32),
            pltpu.VMEM((N,), jnp.float32),
            pltpu.VMEM((N,), jnp.int32),
            pltpu.VMEM((N,), jnp.int32),
            pltpu.VMEM((6 * G,), jnp.float32),
            pltpu.VMEM((N,), jnp.float32),
            pltpu.VMEM((N,), jnp.float32),
            pltpu.VMEM((N,), jnp.float32),
            pltpu.VMEM((N,), jnp.float32),
            pltpu.VMEM((N,), jnp.float32),
            pltpu.SemaphoreType.DMA,
        ],
    )
    def body(boxes_hbm, gtb_hbm, gtf_hbm, gid_hbm, flag_hbm,
             o1_hbm, o2_hbm, o3_hbm, o4_hbm, o5_hbm,
             bx1_v, by1_v, bx2_v, by2_v, gid_v, flag_v, gt_v,
             o1_v, o2_v, o3_v, o4_v, o5_v, sem):
        w = lax.axis_index("s") * NC + lax.axis_index("c")
        cps = [
            pltpu.async_copy(boxes_hbm.at[w, 0], bx1_v, sem),
            pltpu.async_copy(boxes_hbm.at[w, 1], by1_v, sem),
            pltpu.async_copy(boxes_hbm.at[w, 2], bx2_v, sem),
            pltpu.async_copy(boxes_hbm.at[w, 3], by2_v, sem),
            pltpu.async_copy(gid_hbm.at[w], gid_v, sem),
            pltpu.async_copy(flag_hbm.at[w], flag_v, sem),
            pltpu.async_copy(gtb_hbm.at[4, w], gt_v.at[pl.ds(0, G)], sem),
            pltpu.async_copy(gtf_hbm.at[0, w], gt_v.at[pl.ds(1 * G, G)], sem),
            pltpu.async_copy(gtf_hbm.at[1, w], gt_v.at[pl.ds(2 * G, G)], sem),
            pltpu.async_copy(gtf_hbm.at[2, w], gt_v.at[pl.ds(3 * G, G)], sem),
            pltpu.async_copy(gtf_hbm.at[3, w], gt_v.at[pl.ds(4 * G, G)], sem),
            pltpu.async_copy(gtf_hbm.at[8, w], gt_v.at[pl.ds(5 * G, G)], sem),
        ]
        for cp in cps:
            cp.wait()

        def chunk(s):
            gid = gid_v[pl.ds(s, _L)]
            flag = flag_v[pl.ds(s, _L)]
            bx1 = bx1_v[pl.ds(s, _L)]
            by1 = by1_v[pl.ds(s, _L)]
            bx2 = bx2_v[pl.ds(s, _L)]
            by2 = by2_v[pl.ds(s, _L)]
            cls = plsc.load_gather(gt_v, [gid])
            gx1 = plsc.load_gather(gt_v, [gid + G])
            gy1 = plsc.load_gather(gt_v, [gid + 2 * G])
            gx2 = plsc.load_gather(gt_v, [gid + 3 * G])
            gy2 = plsc.load_gather(gt_v, [gid + 4 * G])
            gcl = plsc.load_gather(gt_v, [gid + 5 * G])

            pos = flag > 0
            regm0 = jnp.logical_and(pos, cls > 0.0)
            g = jnp.where(flag == 0, 0.0, gcl)
            g = jnp.where(flag < 0, -1.0, g)
            g = jnp.where(cls == 0.0, -1.0, g)
            clsm = g >= 0.0
            dx = gx1 - gx2
            dy = gy1 - gy2
            dxz = dx == 0.0
            dxs = jnp.where(dxz, 1.0, dx)
            slope = dy / dxs
            cy1 = jnp.where(dxz, 0.0, slope * (bx1 - gx2) + gy2)
            cy2 = jnp.where(dxz, 0.0, slope * (bx2 - gx2) + gy2)
            bw = bx2 - bx1
            bh = by2 - by1
            m = jnp.logical_and(jnp.logical_and(bw > 0.0, bh > 0.0),
                                jnp.logical_not(dxz))
            inv_bh = 1.0 / bh
            r1 = (cy1 - by2) * inv_bh
            r2 = (cy2 - by2) * inv_bh
            regm = jnp.logical_and(regm0, m)

            o1_v[pl.ds(s, _L)] = g
            o2_v[pl.ds(s, _L)] = jnp.where(clsm, 1.0, 0.0)
            o3_v[pl.ds(s, _L)] = r1
            o4_v[pl.ds(s, _L)] = r2
            o5_v[pl.ds(s, _L)] = jnp.where(regm, 1.0, 0.0)

        @plsc.parallel_loop(0, n_full * _L, step=_L, unroll=2)
        def _(s):
            chunk(s)

        if has_tail:
            chunk(N - _L)

        ocps = [
            pltpu.async_copy(o1_v, o1_hbm.at[w], sem),
            pltpu.async_copy(o2_v, o2_hbm.at[w], sem),
            pltpu.async_copy(o3_v, o3_hbm.at[w], sem),
            pltpu.async_copy(o4_v, o4_hbm.at[w], sem),
            pltpu.async_copy(o5_v, o5_hbm.at[w], sem),
        ]
        for cp in ocps:
            cp.wait()

    return body(boxes_t, gtb_t, gtf_t, gid2, flag2)


@jax.jit
def kernel(boxes, gt_boxes, gt_flanks, match_pos_flag, match_gt_id):
    B, N, _ = boxes.shape
    G = gt_boxes.shape[1]

    o_cls, o_clsm, o_r1, o_r2, o_rm = _sc_encode(
        jnp.transpose(boxes, (0, 2, 1)),
        jnp.transpose(gt_boxes, (2, 0, 1)),
        jnp.transpose(gt_flanks, (2, 0, 1)),
        match_gt_id, match_pos_flag, B, N, G)
    gdls_cls = o_cls[..., None]
    cls_label_mask = (o_clsm != 0.0)[..., None]
    reg_label = jnp.stack([o_r1, o_r2], axis=-1)
    reg_label_mask = jnp.broadcast_to((o_rm != 0.0)[..., None],
                                      reg_label.shape)
    return gdls_cls, cls_label_mask, reg_label, reg_label_mask


# speedup vs baseline: 1.0439x; 1.0439x over previous
import functools

import jax
import jax.numpy as jnp
from jax import lax
from jax.experimental import pallas as pl
from jax.experimental.pallas import tpu as pltpu, tpu_sc as plsc

_L = 16


def _sc_encode(boxes_t, gtb_t, gtf_t, gid2, flag2, B, N, G):
    info = plsc.get_sparse_core_info()
    NC, NS = info.num_cores, info.num_subcores
    assert NC * NS == B, (NC, NS, B)
    n_full = N // _L
    has_tail = (N % _L) != 0
    mesh = plsc.VectorSubcoreMesh(core_axis_name="c", subcore_axis_name="s")

    @functools.partial(
        pl.kernel,
        out_type=(
            jax.ShapeDtypeStruct((B, 8, 128), jnp.float32),
            jax.ShapeDtypeStruct((B, N), jnp.float32),
            jax.ShapeDtypeStruct((B, 16, 128), jnp.float32),
            jax.ShapeDtypeStruct((B, N), jnp.float32),
        ),
        mesh=mesh,
        compiler_params=pltpu.CompilerParams(
            needs_layout_passes=False,
            skip_device_barrier=True,
            disable_bounds_checks=True,
            disable_semaphore_checks=True,
        ),
        scratch_types=[
            pltpu.VMEM((N,), jnp.float32),
            pltpu.VMEM((N,), jnp.float32),
            pltpu.VMEM((N,), jnp.float32),
            pltpu.VMEM((N,), jnp.float32),
            pltpu.VMEM((N,), jnp.int32),
            pltpu.VMEM((N,), jnp.int32),
            pltpu.VMEM((6 * G,), jnp.float32),
            pltpu.VMEM((8, 128), jnp.float32),
            pltpu.VMEM((N,), jnp.float32),
            pltpu.VMEM((16, 128), jnp.float32),
            pltpu.VMEM((N,), jnp.float32),
            pltpu.SemaphoreType.DMA,
        ],
    )
    def body(boxes_hbm, gtb_hbm, gtf_hbm, gid_hbm, flag_hbm,
             o1_hbm, o2_hbm, o3_hbm, o5_hbm,
             bx1_v, by1_v, bx2_v, by2_v, gid_v, flag_v, gt_v,
             o1_v, o2_v, o3_v, o5_v, sem):
        w = lax.axis_index("s") * NC + lax.axis_index("c")
        cps = [
            pltpu.async_copy(boxes_hbm.at[w, 0], bx1_v, sem),
            pltpu.async_copy(boxes_hbm.at[w, 1], by1_v, sem),
            pltpu.async_copy(boxes_hbm.at[w, 2], bx2_v, sem),
            pltpu.async_copy(boxes_hbm.at[w, 3], by2_v, sem),
            pltpu.async_copy(gid_hbm.at[w], gid_v, sem),
            pltpu.async_copy(flag_hbm.at[w], flag_v, sem),
            pltpu.async_copy(gtb_hbm.at[4, w], gt_v.at[pl.ds(0, G)], sem),
            pltpu.async_copy(gtf_hbm.at[0, w], gt_v.at[pl.ds(1 * G, G)], sem),
            pltpu.async_copy(gtf_hbm.at[1, w], gt_v.at[pl.ds(2 * G, G)], sem),
            pltpu.async_copy(gtf_hbm.at[2, w], gt_v.at[pl.ds(3 * G, G)], sem),
            pltpu.async_copy(gtf_hbm.at[3, w], gt_v.at[pl.ds(4 * G, G)], sem),
            pltpu.async_copy(gtf_hbm.at[8, w], gt_v.at[pl.ds(5 * G, G)], sem),
        ]
        for cp in cps:
            cp.wait()

        def chunk(s):
            gid = gid_v[pl.ds(s, _L)]
            flag = flag_v[pl.ds(s, _L)]
            bx1 = bx1_v[pl.ds(s, _L)]
            by1 = by1_v[pl.ds(s, _L)]
            bx2 = bx2_v[pl.ds(s, _L)]
            by2 = by2_v[pl.ds(s, _L)]
            cls = plsc.load_gather(gt_v, [gid])
            gx1 = plsc.load_gather(gt_v, [gid + G])
            gy1 = plsc.load_gather(gt_v, [gid + 2 * G])
            gx2 = plsc.load_gather(gt_v, [gid + 3 * G])
            gy2 = plsc.load_gather(gt_v, [gid + 4 * G])
            gcl = plsc.load_gather(gt_v, [gid + 5 * G])

            pos = flag > 0
            regm0 = jnp.logical_and(pos, cls > 0.0)
            g = jnp.where(flag == 0, 0.0, gcl)
            g = jnp.where(flag < 0, -1.0, g)
            g = jnp.where(cls == 0.0, -1.0, g)
            clsm = g >= 0.0
            dx = gx1 - gx2
            dy = gy1 - gy2
            dxz = dx == 0.0
            dxs = jnp.where(dxz, 1.0, dx)
            slope = dy / dxs
            cy1 = jnp.where(dxz, 0.0, slope * (bx1 - gx2) + gy2)
            cy2 = jnp.where(dxz, 0.0, slope * (bx2 - gx2) + gy2)
            bw = bx2 - bx1
            bh = by2 - by1
            m = jnp.logical_and(jnp.logical_and(bw > 0.0, bh > 0.0),
                                jnp.logical_not(dxz))
            inv_bh = 1.0 / bh
            r1 = (cy1 - by2) * inv_bh
            r2 = (cy2 - by2) * inv_bh
            regm = jnp.logical_and(regm0, m)

            blk = jax.lax.shift_right_logical(s, 7)
            col = jnp.bitwise_and(s, 127)
            o1_v[blk, pl.ds(col, _L)] = g
            o2_v[pl.ds(s, _L)] = jnp.where(clsm, 1.0, 0.0)
            o3_v[2 * blk, pl.ds(col, _L)] = r1
            o3_v[2 * blk + 1, pl.ds(col, _L)] = r2
            o5_v[pl.ds(s, _L)] = jnp.where(regm, 1.0, 0.0)

        @plsc.parallel_loop(0, n_full * _L, step=_L)
        def _(s):
            chunk(s)

        if has_tail:
            chunk(N - _L)

        ocps = [
            pltpu.async_copy(o1_v, o1_hbm.at[w], sem),
            pltpu.async_copy(o2_v, o2_hbm.at[w], sem),
            pltpu.async_copy(o3_v, o3_hbm.at[w], sem),
            pltpu.async_copy(o5_v, o5_hbm.at[w], sem),
        ]
        for cp in ocps:
            cp.wait()

    return body(boxes_t, gtb_t, gtf_t, gid2, flag2)


@jax.jit
def kernel(boxes, gt_boxes, gt_flanks, match_pos_flag, match_gt_id):
    B, N, _ = boxes.shape
    G = gt_boxes.shape[1]

    o_cls, o_clsm, o_reg, o_rm = _sc_encode(
        jnp.transpose(boxes, (0, 2, 1)),
        jnp.transpose(gt_boxes, (2, 0, 1)),
        jnp.transpose(gt_flanks, (2, 0, 1)),
        match_gt_id, match_pos_flag, B, N, G)
    gdls_cls = o_cls.reshape(B, 1024)[:, :N, None]
    cls_label_mask = (o_clsm != 0.0)[..., None]
    reg_label = jnp.transpose(o_reg.reshape(B, 8, 2, 128),
                              (0, 1, 3, 2)).reshape(B, 1024, 2)[:, :N]
    reg_label_mask = jnp.broadcast_to((o_rm != 0.0)[..., None],
                                      reg_label.shape)
    return gdls_cls, cls_label_mask, reg_label, reg_label_mask


# speedup vs baseline: 1.0490x; 1.0049x over previous
import functools

import jax
import jax.numpy as jnp
from jax import lax
from jax.experimental import pallas as pl
from jax.experimental.pallas import tpu as pltpu, tpu_sc as plsc

_L = 16


def _sc_encode(boxes_t, gtb_t, gtf_t, gid2, flag2, B, N, G):
    info = plsc.get_sparse_core_info()
    NC, NS = info.num_cores, info.num_subcores
    assert NC * NS == B, (NC, NS, B)
    n_full = N // _L
    has_tail = (N % _L) != 0
    NT = (N + 127) // 128
    TPAD = NT * 128
    mesh = plsc.VectorSubcoreMesh(core_axis_name="c", subcore_axis_name="s")

    @functools.partial(
        pl.kernel,
        out_type=(
            jax.ShapeDtypeStruct((B, NT, 128), jnp.float32),
            jax.ShapeDtypeStruct((B, N), jnp.float32),
            jax.ShapeDtypeStruct((B, 2 * NT, 128), jnp.float32),
            jax.ShapeDtypeStruct((B, N), jnp.float32),
        ),
        mesh=mesh,
        compiler_params=pltpu.CompilerParams(
            needs_layout_passes=False,
            skip_device_barrier=True,
            disable_bounds_checks=True,
            disable_semaphore_checks=True,
        ),
        scratch_types=[
            pltpu.VMEM((N,), jnp.float32),
            pltpu.VMEM((N,), jnp.float32),
            pltpu.VMEM((N,), jnp.float32),
            pltpu.VMEM((N,), jnp.float32),
            pltpu.VMEM((N,), jnp.int32),
            pltpu.VMEM((N,), jnp.int32),
            pltpu.VMEM((6 * G,), jnp.float32),
            pltpu.VMEM((NT, 128), jnp.float32),
            pltpu.VMEM((N,), jnp.float32),
            pltpu.VMEM((2 * NT, 128), jnp.float32),
            pltpu.VMEM((N,), jnp.float32),
            pltpu.SemaphoreType.DMA,
        ],
    )
    def body(boxes_hbm, gtb_hbm, gtf_hbm, gid_hbm, flag_hbm,
             o1_hbm, o2_hbm, o3_hbm, o5_hbm,
             bx1_v, by1_v, bx2_v, by2_v, gid_v, flag_v, gt_v,
             o1_v, o2_v, o3_v, o5_v, sem):
        w = lax.axis_index("s") * NC + lax.axis_index("c")
        cps = [
            pltpu.async_copy(boxes_hbm.at[w, 0], bx1_v, sem),
            pltpu.async_copy(boxes_hbm.at[w, 1], by1_v, sem),
            pltpu.async_copy(boxes_hbm.at[w, 2], bx2_v, sem),
            pltpu.async_copy(boxes_hbm.at[w, 3], by2_v, sem),
            pltpu.async_copy(gid_hbm.at[w], gid_v, sem),
            pltpu.async_copy(flag_hbm.at[w], flag_v, sem),
            pltpu.async_copy(gtb_hbm.at[4, w], gt_v.at[pl.ds(0, G)], sem),
            pltpu.async_copy(gtf_hbm.at[0, w], gt_v.at[pl.ds(1 * G, G)], sem),
            pltpu.async_copy(gtf_hbm.at[1, w], gt_v.at[pl.ds(2 * G, G)], sem),
            pltpu.async_copy(gtf_hbm.at[2, w], gt_v.at[pl.ds(3 * G, G)], sem),
            pltpu.async_copy(gtf_hbm.at[3, w], gt_v.at[pl.ds(4 * G, G)], sem),
            pltpu.async_copy(gtf_hbm.at[8, w], gt_v.at[pl.ds(5 * G, G)], sem),
        ]
        for cp in cps:
            cp.wait()

        def chunk(s):
            gid = gid_v[pl.ds(s, _L)]
            flag = flag_v[pl.ds(s, _L)]
            bx1 = bx1_v[pl.ds(s, _L)]
            by1 = by1_v[pl.ds(s, _L)]
            bx2 = bx2_v[pl.ds(s, _L)]
            by2 = by2_v[pl.ds(s, _L)]
            cls = plsc.load_gather(gt_v, [gid])
            gx1 = plsc.load_gather(gt_v, [gid + G])
            gy1 = plsc.load_gather(gt_v, [gid + 2 * G])
            gx2 = plsc.load_gather(gt_v, [gid + 3 * G])
            gy2 = plsc.load_gather(gt_v, [gid + 4 * G])
            gcl = plsc.load_gather(gt_v, [gid + 5 * G])

            pos = flag > 0
            regm0 = jnp.logical_and(pos, cls > 0.0)
            g = jnp.where(flag == 0, 0.0, gcl)
            g = jnp.where(flag < 0, -1.0, g)
            g = jnp.where(cls == 0.0, -1.0, g)
            clsm = g >= 0.0
            dx = gx1 - gx2
            dy = gy1 - gy2
            dxz = dx == 0.0
            dxs = jnp.where(dxz, 1.0, dx)
            slope = dy / dxs
            cy1 = jnp.where(dxz, 0.0, slope * (bx1 - gx2) + gy2)
            cy2 = jnp.where(dxz, 0.0, slope * (bx2 - gx2) + gy2)
            bw = bx2 - bx1
            bh = by2 - by1
            m = jnp.logical_and(jnp.logical_and(bw > 0.0, bh > 0.0),
                                jnp.logical_not(dxz))
            inv_bh = 1.0 / bh
            r1 = (cy1 - by2) * inv_bh
            r2 = (cy2 - by2) * inv_bh
            regm = jnp.logical_and(regm0, m)

            blk = jax.lax.shift_right_logical(s, 7)
            col = jnp.bitwise_and(s, 127)
            o1_v[blk, pl.ds(col, _L)] = g
            o2_v[pl.ds(s, _L)] = jnp.where(clsm, 1.0, 0.0)
            o3_v[2 * blk, pl.ds(col, _L)] = r1
            o3_v[2 * blk + 1, pl.ds(col, _L)] = r2
            o5_v[pl.ds(s, _L)] = jnp.where(regm, 1.0, 0.0)

        @plsc.parallel_loop(0, n_full * _L, step=_L)
        def _(s):
            chunk(s)

        if has_tail:
            chunk(N - _L)

        ocps = [
            pltpu.async_copy(o1_v, o1_hbm.at[w], sem),
            pltpu.async_copy(o2_v, o2_hbm.at[w], sem),
            pltpu.async_copy(o3_v, o3_hbm.at[w], sem),
            pltpu.async_copy(o5_v, o5_hbm.at[w], sem),
        ]
        for cp in ocps:
            cp.wait()

    return body(boxes_t, gtb_t, gtf_t, gid2, flag2)


@jax.jit
def kernel(boxes, gt_boxes, gt_flanks, match_pos_flag, match_gt_id):
    B, N, _ = boxes.shape
    G = gt_boxes.shape[1]

    o_cls, o_clsm, o_reg, o_rm = _sc_encode(
        jnp.transpose(boxes, (0, 2, 1)),
        jnp.transpose(gt_boxes, (2, 0, 1)),
        jnp.transpose(gt_flanks, (2, 0, 1)),
        match_gt_id, match_pos_flag, B, N, G)
    TPAD = o_cls.shape[1] * 128
    gdls_cls = o_cls.reshape(B, TPAD)[:, :N, None]
    cls_label_mask = (o_clsm != 0.0)[..., None]
    reg_label = jnp.transpose(o_reg.reshape(B, TPAD // 128, 2, 128),
                              (0, 1, 3, 2)).reshape(B, TPAD, 2)[:, :N]
    reg_label_mask = jnp.broadcast_to((o_rm != 0.0)[..., None],
                                      reg_label.shape)
    return gdls_cls, cls_label_mask, reg_label, reg_label_mask


# speedup vs baseline: 1.0491x; 1.0000x over previous
import functools

import jax
import jax.numpy as jnp
from jax import lax
from jax.experimental import pallas as pl
from jax.experimental.pallas import tpu as pltpu, tpu_sc as plsc

_L = 16


def _sc_encode(boxes_t, gtb_t, gtf_t, gid2, flag2, B, N, G):
    info = plsc.get_sparse_core_info()
    NC, NS = info.num_cores, info.num_subcores
    assert NC * NS == B, (NC, NS, B)
    n_full = N // _L
    has_tail = (N % _L) != 0
    NT = (N + 127) // 128
    TPAD = NT * 128
    mesh = plsc.VectorSubcoreMesh(core_axis_name="c", subcore_axis_name="s")

    @functools.partial(
        pl.kernel,
        out_type=(
            jax.ShapeDtypeStruct((B, NT, 128), jnp.float32),
            jax.ShapeDtypeStruct((B, 2 * NT, 128), jnp.float32),
            jax.ShapeDtypeStruct((B, N), jnp.float32),
        ),
        mesh=mesh,
        compiler_params=pltpu.CompilerParams(
            needs_layout_passes=False,
            skip_device_barrier=True,
            disable_bounds_checks=True,
            disable_semaphore_checks=True,
        ),
        scratch_types=[
            pltpu.VMEM((N,), jnp.float32),
            pltpu.VMEM((N,), jnp.float32),
            pltpu.VMEM((N,), jnp.float32),
            pltpu.VMEM((N,), jnp.float32),
            pltpu.VMEM((N,), jnp.int32),
            pltpu.VMEM((N,), jnp.int32),
            pltpu.VMEM((6 * G,), jnp.float32),
            pltpu.VMEM((NT, 128), jnp.float32),
            pltpu.VMEM((2 * NT, 128), jnp.float32),
            pltpu.VMEM((N,), jnp.float32),
            pltpu.SemaphoreType.DMA,
        ],
    )
    def body(boxes_hbm, gtb_hbm, gtf_hbm, gid_hbm, flag_hbm,
             o1_hbm, o3_hbm, o5_hbm,
             bx1_v, by1_v, bx2_v, by2_v, gid_v, flag_v, gt_v,
             o1_v, o3_v, o5_v, sem):
        w = lax.axis_index("s") * NC + lax.axis_index("c")
        cps = [
            pltpu.async_copy(boxes_hbm.at[w, 0], bx1_v, sem),
            pltpu.async_copy(boxes_hbm.at[w, 1], by1_v, sem),
            pltpu.async_copy(boxes_hbm.at[w, 2], bx2_v, sem),
            pltpu.async_copy(boxes_hbm.at[w, 3], by2_v, sem),
            pltpu.async_copy(gid_hbm.at[w], gid_v, sem),
            pltpu.async_copy(flag_hbm.at[w], flag_v, sem),
            pltpu.async_copy(gtb_hbm.at[4, w], gt_v.at[pl.ds(0, G)], sem),
            pltpu.async_copy(gtf_hbm.at[0, w], gt_v.at[pl.ds(1 * G, G)], sem),
            pltpu.async_copy(gtf_hbm.at[1, w], gt_v.at[pl.ds(2 * G, G)], sem),
            pltpu.async_copy(gtf_hbm.at[2, w], gt_v.at[pl.ds(3 * G, G)], sem),
            pltpu.async_copy(gtf_hbm.at[3, w], gt_v.at[pl.ds(4 * G, G)], sem),
            pltpu.async_copy(gtf_hbm.at[8, w], gt_v.at[pl.ds(5 * G, G)], sem),
        ]
        for cp in cps:
            cp.wait()

        def chunk(s):
            gid = gid_v[pl.ds(s, _L)]
            flag = flag_v[pl.ds(s, _L)]
            bx1 = bx1_v[pl.ds(s, _L)]
            by1 = by1_v[pl.ds(s, _L)]
            bx2 = bx2_v[pl.ds(s, _L)]
            by2 = by2_v[pl.ds(s, _L)]
            cls = plsc.load_gather(gt_v, [gid])
            gx1 = plsc.load_gather(gt_v, [gid + G])
            gy1 = plsc.load_gather(gt_v, [gid + 2 * G])
            gx2 = plsc.load_gather(gt_v, [gid + 3 * G])
            gy2 = plsc.load_gather(gt_v, [gid + 4 * G])
            gcl = plsc.load_gather(gt_v, [gid + 5 * G])

            pos = flag > 0
            regm0 = jnp.logical_and(pos, cls > 0.0)
            g = jnp.where(flag == 0, 0.0, gcl)
            g = jnp.where(flag < 0, -1.0, g)
            g = jnp.where(cls == 0.0, -1.0, g)
            dx = gx1 - gx2
            dy = gy1 - gy2
            dxz = dx == 0.0
            dxs = jnp.where(dxz, 1.0, dx)
            slope = dy / dxs
            cy1 = jnp.where(dxz, 0.0, slope * (bx1 - gx2) + gy2)
            cy2 = jnp.where(dxz, 0.0, slope * (bx2 - gx2) + gy2)
            bw = bx2 - bx1
            bh = by2 - by1
            m = jnp.logical_and(jnp.logical_and(bw > 0.0, bh > 0.0),
                                jnp.logical_not(dxz))
            inv_bh = 1.0 / bh
            r1 = (cy1 - by2) * inv_bh
            r2 = (cy2 - by2) * inv_bh
            regm = jnp.logical_and(regm0, m)

            blk = jax.lax.shift_right_logical(s, 7)
            col = jnp.bitwise_and(s, 127)
            o1_v[blk, pl.ds(col, _L)] = g
            o3_v[2 * blk, pl.ds(col, _L)] = r1
            o3_v[2 * blk + 1, pl.ds(col, _L)] = r2
            o5_v[pl.ds(s, _L)] = jnp.where(regm, 1.0, 0.0)

        @plsc.parallel_loop(0, n_full * _L, step=_L)
        def _(s):
            chunk(s)

        if has_tail:
            chunk(N - _L)

        ocps = [
            pltpu.async_copy(o1_v, o1_hbm.at[w], sem),
            pltpu.async_copy(o3_v, o3_hbm.at[w], sem),
            pltpu.async_copy(o5_v, o5_hbm.at[w], sem),
        ]
        for cp in ocps:
            cp.wait()

    return body(boxes_t, gtb_t, gtf_t, gid2, flag2)


@jax.jit
def kernel(boxes, gt_boxes, gt_flanks, match_pos_flag, match_gt_id):
    B, N, _ = boxes.shape
    G = gt_boxes.shape[1]

    o_cls, o_reg, o_rm = _sc_encode(
        jnp.transpose(boxes, (0, 2, 1)),
        jnp.transpose(gt_boxes, (2, 0, 1)),
        jnp.transpose(gt_flanks, (2, 0, 1)),
        match_gt_id, match_pos_flag, B, N, G)
    TPAD = o_cls.shape[1] * 128
    gdls_cls = o_cls.reshape(B, TPAD)[:, :N, None]
    cls_label_mask = gdls_cls >= 0.0
    reg_label = jnp.transpose(o_reg.reshape(B, TPAD // 128, 2, 128),
                              (0, 1, 3, 2)).reshape(B, TPAD, 2)[:, :N]
    reg_label_mask = jnp.broadcast_to((o_rm != 0.0)[..., None],
                                      reg_label.shape)
    return gdls_cls, cls_label_mask, reg_label, reg_label_mask


# speedup vs baseline: 1.0784x; 1.0279x over previous
import functools

import jax
import jax.numpy as jnp
from jax import lax
from jax.experimental import pallas as pl
from jax.experimental.pallas import tpu as pltpu, tpu_sc as plsc

_L = 16


def _sc_encode(boxes_t, gtb_t, gtf_t, gid2, flag2, B, N, G):
    info = plsc.get_sparse_core_info()
    NC, NS = info.num_cores, info.num_subcores
    assert NC * NS == B, (NC, NS, B)
    n_full = N // _L
    has_tail = (N % _L) != 0
    NT = (N + 127) // 128
    TPAD = NT * 128
    mesh = plsc.VectorSubcoreMesh(core_axis_name="c", subcore_axis_name="s")

    @functools.partial(
        pl.kernel,
        out_type=(
            jax.ShapeDtypeStruct((B, NT, 128), jnp.float32),
            jax.ShapeDtypeStruct((B, 2 * NT, 128), jnp.float32),
            jax.ShapeDtypeStruct((B, N), jnp.float32),
        ),
        mesh=mesh,
        compiler_params=pltpu.CompilerParams(
            needs_layout_passes=False,
            skip_device_barrier=True,
            disable_bounds_checks=True,
            disable_semaphore_checks=True,
        ),
        scratch_types=[
            pltpu.VMEM((N,), jnp.float32),
            pltpu.VMEM((N,), jnp.float32),
            pltpu.VMEM((N,), jnp.float32),
            pltpu.VMEM((N,), jnp.float32),
            pltpu.VMEM((N,), jnp.int32),
            pltpu.VMEM((N,), jnp.int32),
            pltpu.VMEM((6 * G,), jnp.float32),
            pltpu.VMEM((NT, 128), jnp.float32),
            pltpu.VMEM((2 * NT, 128), jnp.float32),
            pltpu.VMEM((N,), jnp.float32),
            pltpu.SemaphoreType.DMA,
        ],
    )
    def body(boxes_hbm, gtb_hbm, gtf_hbm, gid_hbm, flag_hbm,
             o1_hbm, o3_hbm, o5_hbm,
             bx1_v, by1_v, bx2_v, by2_v, gid_v, flag_v, gt_v,
             o1_v, o3_v, o5_v, sem):
        w = lax.axis_index("s") * NC + lax.axis_index("c")
        cps = [
            pltpu.async_copy(boxes_hbm.at[w, 0], bx1_v, sem),
            pltpu.async_copy(boxes_hbm.at[w, 1], by1_v, sem),
            pltpu.async_copy(boxes_hbm.at[w, 2], bx2_v, sem),
            pltpu.async_copy(boxes_hbm.at[w, 3], by2_v, sem),
            pltpu.async_copy(gid_hbm.at[w], gid_v, sem),
            pltpu.async_copy(flag_hbm.at[w], flag_v, sem),
            pltpu.async_copy(gtb_hbm.at[4, w], gt_v.at[pl.ds(0, G)], sem),
            pltpu.async_copy(gtf_hbm.at[0, w], gt_v.at[pl.ds(1 * G, G)], sem),
            pltpu.async_copy(gtf_hbm.at[1, w], gt_v.at[pl.ds(2 * G, G)], sem),
            pltpu.async_copy(gtf_hbm.at[2, w], gt_v.at[pl.ds(3 * G, G)], sem),
            pltpu.async_copy(gtf_hbm.at[3, w], gt_v.at[pl.ds(4 * G, G)], sem),
            pltpu.async_copy(gtf_hbm.at[8, w], gt_v.at[pl.ds(5 * G, G)], sem),
        ]
        for cp in cps:
            cp.wait()

        def chunk(s):
            gid = gid_v[pl.ds(s, _L)]
            flag = flag_v[pl.ds(s, _L)]
            bx1 = bx1_v[pl.ds(s, _L)]
            by1 = by1_v[pl.ds(s, _L)]
            bx2 = bx2_v[pl.ds(s, _L)]
            by2 = by2_v[pl.ds(s, _L)]
            cls = plsc.load_gather(gt_v, [gid])
            gx1 = plsc.load_gather(gt_v, [gid + G])
            gy1 = plsc.load_gather(gt_v, [gid + 2 * G])
            gx2 = plsc.load_gather(gt_v, [gid + 3 * G])
            gy2 = plsc.load_gather(gt_v, [gid + 4 * G])
            gcl = plsc.load_gather(gt_v, [gid + 5 * G])

            pos = flag > 0
            regm0 = jnp.logical_and(pos, cls > 0.0)
            g = jnp.where(flag == 0, 0.0, gcl)
            g = jnp.where(flag < 0, -1.0, g)
            g = jnp.where(cls == 0.0, -1.0, g)
            dx = gx1 - gx2
            dy = gy1 - gy2
            dxz = dx == 0.0
            dxs = jnp.where(dxz, 1.0, dx)
            slope = dy / dxs
            cy1 = jnp.where(dxz, 0.0, slope * (bx1 - gx2) + gy2)
            cy2 = jnp.where(dxz, 0.0, slope * (bx2 - gx2) + gy2)
            bw = bx2 - bx1
            bh = by2 - by1
            m = jnp.logical_and(jnp.logical_and(bw > 0.0, bh > 0.0),
                                jnp.logical_not(dxz))
            inv_bh = 1.0 / bh
            r1 = (cy1 - by2) * inv_bh
            r2 = (cy2 - by2) * inv_bh
            regm = jnp.logical_and(regm0, m)

            blk = jax.lax.shift_right_logical(s, 7)
            col = jnp.bitwise_and(s, 127)
            o1_v[blk, pl.ds(col, _L)] = g
            o3_v[2 * blk, pl.ds(col, _L)] = r1
            o3_v[2 * blk + 1, pl.ds(col, _L)] = r2
            o5_v[pl.ds(s, _L)] = jnp.where(regm, 1.0, 0.0)

        @plsc.parallel_loop(0, n_full * _L, step=_L)
        def _(s):
            chunk(s)

        if has_tail:
            chunk(N - _L)

        ocps = [
            pltpu.async_copy(o1_v, o1_hbm.at[w], sem),
            pltpu.async_copy(o3_v, o3_hbm.at[w], sem),
            pltpu.async_copy(o5_v, o5_hbm.at[w], sem),
        ]
        for cp in ocps:
            cp.wait()

    return body(boxes_t, gtb_t, gtf_t, gid2, flag2)


@jax.jit
def kernel(boxes, gt_boxes, gt_flanks, match_pos_flag, match_gt_id):
    B, N, _ = boxes.shape
    G = gt_boxes.shape[1]

    o_cls, o_reg, o_rm = _sc_encode(
        jnp.transpose(boxes, (0, 2, 1)),
        jnp.transpose(gt_boxes, (2, 0, 1)),
        jnp.transpose(gt_flanks, (2, 0, 1)),
        match_gt_id, match_pos_flag, B, N, G)
    TPAD = o_cls.shape[1] * 128
    gdls_cls = o_cls.reshape(B, TPAD, 1)[:, :N]
    cls_label_mask = gdls_cls >= 0.0
    reg_label = jnp.transpose(o_reg.reshape(B, TPAD // 128, 2, 128),
                              (0, 1, 3, 2)).reshape(B, TPAD, 2)[:, :N]
    reg_label_mask = jnp.broadcast_to((o_rm != 0.0)[..., None],
                                      reg_label.shape)
    return gdls_cls, cls_label_mask, reg_label, reg_label_mask
